# Initial kernel scaffold; baseline (speedup 1.0000x reference)
import jax
import jax.numpy as jnp
from jax.experimental import pallas as pl


def kernel(input, coordinates):
    raise NotImplementedError("write your pallas kernel here")


# speedup vs baseline: 2.4187x; 2.4187x over previous
import functools
import math

import jax
import jax.numpy as jnp
from jax import lax
from jax.experimental import pallas as pl
from jax.experimental.pallas import tpu as pltpu
from jax.experimental.pallas import tpu_sc as plsc

_NUM_CORES = 2
_NUM_SUBCORES = 16
_NUM_WORKERS = _NUM_CORES * _NUM_SUBCORES
_LANES = 16

_CHUNK = 128


def _make_gather(B, V, D, n_chunks):
    b_per_w = n_chunks * _CHUNK
    scale = math.sqrt(D)
    mesh = plsc.VectorSubcoreMesh(
        core_axis_name="c", subcore_axis_name="s"
    )

    @functools.partial(
        pl.kernel,
        out_type=jax.ShapeDtypeStruct((B, D), jnp.float32),
        mesh=mesh,
        scratch_types=[
            pltpu.VMEM((n_chunks, _CHUNK), jnp.int32),
            pltpu.VMEM((_CHUNK, D), jnp.float32),
            pltpu.SemaphoreType.DMA,
        ],
    )
    def gather_kernel(idx_hbm, table_hbm, out_hbm, idx_v, rows_v, sem):
        wid = lax.axis_index("s") * _NUM_CORES + lax.axis_index("c")
        base = wid * b_per_w
        pltpu.sync_copy(idx_hbm.at[wid], idx_v)

        def chunk_body(j, carry):
            pltpu.async_copy(table_hbm.at[idx_v.at[j]], rows_v, sem).wait()

            def scale_body(r, c):
                for k in range(D // _LANES):
                    sl = pl.ds(k * _LANES, _LANES)
                    rows_v[r, sl] = rows_v[r, sl] * scale
                return c

            lax.fori_loop(0, _CHUNK, scale_body, 0, unroll=2)
            pltpu.sync_copy(
                rows_v, out_hbm.at[pl.ds(base + j * _CHUNK, _CHUNK)]
            )
            return carry

        lax.fori_loop(0, n_chunks, chunk_body, 0)

    return gather_kernel


@jax.jit
def kernel(input, coordinates):
    V, D = coordinates.shape
    B = input.size
    b_per_w = B // _NUM_WORKERS
    n_chunks = b_per_w // _CHUNK
    idx = input.reshape(_NUM_WORKERS, n_chunks, _CHUNK).astype(jnp.int32)
    out = _make_gather(B, V, D, n_chunks)(idx, coordinates)
    return out.reshape(input.shape + (D,))


# speedup vs baseline: 2.9556x; 1.2220x over previous
import functools
import math

import jax
import jax.numpy as jnp
from jax import lax
from jax.experimental import pallas as pl
from jax.experimental.pallas import tpu as pltpu
from jax.experimental.pallas import tpu_sc as plsc

_NUM_CORES = 2
_NUM_SUBCORES = 16
_NUM_WORKERS = _NUM_CORES * _NUM_SUBCORES
_LANES = 16

_CHUNK = 128
_NBUF = 5
_LEAD = 3


def _make_gather(B, V, D, n_chunks):
    b_per_w = n_chunks * _CHUNK
    n_groups = n_chunks // _NBUF
    scale = math.sqrt(D)
    mesh = plsc.VectorSubcoreMesh(core_axis_name="c", subcore_axis_name="s")

    @functools.partial(
        pl.kernel,
        out_type=jax.ShapeDtypeStruct((B, D), jnp.float32),
        mesh=mesh,
        scratch_types=[
            pltpu.VMEM((n_chunks, _CHUNK), jnp.int32),
            [pltpu.VMEM((_CHUNK, D), jnp.float32) for _ in range(_NBUF)],
            [pltpu.SemaphoreType.DMA for _ in range(_NBUF)],
            [pltpu.SemaphoreType.DMA for _ in range(_NBUF)],
        ],
    )
    def gather_kernel(idx_hbm, table_hbm, out_hbm, idx_v, bufs, gsems, ssems):
        wid = lax.axis_index("s") * _NUM_CORES + lax.axis_index("c")
        base = wid * b_per_w
        pltpu.sync_copy(idx_hbm.at[wid], idx_v)

        for b in range(_LEAD):
            pltpu.async_copy(table_hbm.at[idx_v.at[b]], bufs[b], gsems[b])

        def group_body(g, carry):
            for b in range(_NBUF):
                t = g * _NBUF + b
                pltpu.make_async_copy(
                    table_hbm.at[idx_v.at[t]], bufs[b], gsems[b]
                ).wait()

                @plsc.parallel_loop(0, _CHUNK, unroll=4)
                def scale_rows(r):
                    for k in range(D // _LANES):
                        sl = pl.ds(k * _LANES, _LANES)
                        bufs[b][r, sl] = bufs[b][r, sl] * scale

                pltpu.async_copy(
                    bufs[b],
                    out_hbm.at[pl.ds(base + t * _CHUNK, _CHUNK)],
                    ssems[b],
                )

                bb = (b + _LEAD) % _NBUF

                @pl.when(t >= _NBUF - _LEAD)
                def _():
                    pltpu.make_async_copy(
                        bufs[bb], out_hbm.at[pl.ds(0, _CHUNK)], ssems[bb]
                    ).wait()

                @pl.when(t + _LEAD < n_chunks)
                def _():
                    pltpu.async_copy(
                        table_hbm.at[idx_v.at[t + _LEAD]], bufs[bb], gsems[bb]
                    )

            return carry

        lax.fori_loop(0, n_groups, group_body, 0)

        for t in range(n_chunks - (_NBUF - _LEAD), n_chunks):
            b = t % _NBUF
            pltpu.make_async_copy(
                bufs[b], out_hbm.at[pl.ds(0, _CHUNK)], ssems[b]
            ).wait()

    return gather_kernel


@jax.jit
def kernel(input, coordinates):
    V, D = coordinates.shape
    B = input.size
    b_per_w = B // _NUM_WORKERS
    n_chunks = b_per_w // _CHUNK
    idx = input.reshape(_NUM_WORKERS, n_chunks, _CHUNK).astype(jnp.int32)
    out = _make_gather(B, V, D, n_chunks)(idx, coordinates)
    return out.reshape(input.shape + (D,))


# speedup vs baseline: 5.2525x; 1.7771x over previous
import functools
import math

import jax
import jax.numpy as jnp
from jax import lax
from jax.experimental import pallas as pl
from jax.experimental.pallas import tpu as pltpu
from jax.experimental.pallas import tpu_sc as plsc

_NUM_CORES = 2
_NUM_SUBCORES = 16
_NUM_WORKERS = _NUM_CORES * _NUM_SUBCORES
_LANES = 16

_ROWS_PER_CHUNK = 2
_NBUF = 8
_LEAD = 4


def _make_gather(NB, S, V, D):
    b_per_w = NB // _NUM_WORKERS
    n_chunks = b_per_w // _ROWS_PER_CHUNK
    chunk = _ROWS_PER_CHUNK * S
    n_groups = n_chunks // _NBUF
    scale = math.sqrt(D)
    mesh = plsc.VectorSubcoreMesh(core_axis_name="c", subcore_axis_name="s")

    @functools.partial(
        pl.kernel,
        out_type=jax.ShapeDtypeStruct((NB, S, D), jnp.float32),
        mesh=mesh,
        scratch_types=[
            pltpu.VMEM((n_chunks, chunk), jnp.int32),
            [pltpu.VMEM((chunk, D), jnp.float32) for _ in range(_NBUF)],
            [pltpu.SemaphoreType.DMA for _ in range(_NBUF)],
            [pltpu.SemaphoreType.DMA for _ in range(_NBUF)],
        ],
    )
    def gather_kernel(idx_hbm, table_hbm, out_hbm, idx_v, bufs, gsems, ssems):
        wid = lax.axis_index("s") * _NUM_CORES + lax.axis_index("c")
        b_base = wid * b_per_w
        pltpu.sync_copy(idx_hbm.at[wid], idx_v)

        def fire_gather(t, b):
            pltpu.async_copy(table_hbm.at[idx_v.at[t]], bufs[b], gsems[b])

        def wait_scatters(b):
            for _ in range(_ROWS_PER_CHUNK):
                pltpu.make_async_copy(
                    bufs[b].at[pl.ds(0, S)], out_hbm.at[0], ssems[b]
                ).wait()

        for b in range(_LEAD):
            fire_gather(b, b)

        def group_body(g, carry):
            for b in range(_NBUF):
                t = g * _NBUF + b
                pltpu.make_async_copy(
                    table_hbm.at[idx_v.at[t]], bufs[b], gsems[b]
                ).wait()

                @plsc.parallel_loop(0, chunk, unroll=4)
                def scale_rows(r):
                    for k in range(D // _LANES):
                        sl = pl.ds(k * _LANES, _LANES)
                        bufs[b][r, sl] = bufs[b][r, sl] * scale

                for i in range(_ROWS_PER_CHUNK):
                    pltpu.async_copy(
                        bufs[b].at[pl.ds(i * S, S)],
                        out_hbm.at[b_base + t * _ROWS_PER_CHUNK + i],
                        ssems[b],
                    )

                bb = (b + _LEAD) % _NBUF

                @pl.when(t >= _NBUF - _LEAD)
                def _():
                    wait_scatters(bb)

                @pl.when(t + _LEAD < n_chunks)
                def _():
                    fire_gather(t + _LEAD, bb)

            return carry

        lax.fori_loop(0, n_groups, group_body, 0)

        for t in range(n_chunks - (_NBUF - _LEAD), n_chunks):
            wait_scatters(t % _NBUF)

    return gather_kernel


@jax.jit
def kernel(input, coordinates):
    V, D = coordinates.shape
    NB, S = input.shape
    b_per_w = NB // _NUM_WORKERS
    n_chunks = b_per_w // _ROWS_PER_CHUNK
    idx = input.reshape(
        _NUM_WORKERS, n_chunks, _ROWS_PER_CHUNK * S
    ).astype(jnp.int32)
    return _make_gather(NB, S, V, D)(idx, coordinates)


# speedup vs baseline: 5.2700x; 1.0033x over previous
import functools
import math

import jax
import jax.numpy as jnp
from jax import lax
from jax.experimental import pallas as pl
from jax.experimental.pallas import tpu as pltpu
from jax.experimental.pallas import tpu_sc as plsc

_NUM_CORES = 2
_NUM_SUBCORES = 16
_NUM_WORKERS = _NUM_CORES * _NUM_SUBCORES
_LANES = 16

_ROWS_PER_CHUNK = 2
_NBUF = 8
_LEAD = 4


def _make_gather(NB, S, V, D):
    b_per_w = NB // _NUM_WORKERS
    n_chunks = b_per_w // _ROWS_PER_CHUNK
    chunk = _ROWS_PER_CHUNK * S
    n_groups = n_chunks // _NBUF
    scale = math.sqrt(D)
    mesh = plsc.VectorSubcoreMesh(core_axis_name="c", subcore_axis_name="s")

    @functools.partial(
        pl.kernel,
        out_type=jax.ShapeDtypeStruct((NB, S, D), jnp.float32),
        mesh=mesh,
        compiler_params=pltpu.CompilerParams(use_tc_tiling_on_sc=True),
        scratch_types=[
            pltpu.VMEM((n_chunks, chunk), jnp.int32),
            [pltpu.VMEM((chunk, D), jnp.float32) for _ in range(_NBUF)],
            [pltpu.SemaphoreType.DMA for _ in range(_NBUF)],
            [pltpu.SemaphoreType.DMA for _ in range(_NBUF)],
        ],
    )
    def gather_kernel(idx_hbm, table_hbm, out_hbm, idx_v, bufs, gsems, ssems):
        wid = lax.axis_index("s") * _NUM_CORES + lax.axis_index("c")
        b_base = wid * b_per_w
        pltpu.sync_copy(idx_hbm.at[wid], idx_v)

        def fire_gather(t, b):
            pltpu.async_copy(table_hbm.at[idx_v.at[t]], bufs[b], gsems[b])

        def wait_scatters(b):
            for _ in range(_ROWS_PER_CHUNK):
                pltpu.make_async_copy(
                    bufs[b].at[pl.ds(0, S)], out_hbm.at[0], ssems[b]
                ).wait()

        for b in range(_LEAD):
            fire_gather(b, b)

        def group_body(g, carry):
            for b in range(_NBUF):
                t = g * _NBUF + b
                pltpu.make_async_copy(
                    table_hbm.at[idx_v.at[t]], bufs[b], gsems[b]
                ).wait()

                @plsc.parallel_loop(0, chunk, unroll=4)
                def scale_rows(r):
                    for k in range(D // _LANES):
                        sl = pl.ds(k * _LANES, _LANES)
                        bufs[b][r, sl] = bufs[b][r, sl] * scale

                for i in range(_ROWS_PER_CHUNK):
                    pltpu.async_copy(
                        bufs[b].at[pl.ds(i * S, S)],
                        out_hbm.at[b_base + t * _ROWS_PER_CHUNK + i],
                        ssems[b],
                    )

                bb = (b + _LEAD) % _NBUF

                @pl.when(t >= _NBUF - _LEAD)
                def _():
                    wait_scatters(bb)

                @pl.when(t + _LEAD < n_chunks)
                def _():
                    fire_gather(t + _LEAD, bb)

            return carry

        lax.fori_loop(0, n_groups, group_body, 0)

        for t in range(n_chunks - (_NBUF - _LEAD), n_chunks):
            wait_scatters(t % _NBUF)

    return gather_kernel


@jax.jit
def kernel(input, coordinates):
    V, D = coordinates.shape
    NB, S = input.shape
    b_per_w = NB // _NUM_WORKERS
    n_chunks = b_per_w // _ROWS_PER_CHUNK
    idx = input.reshape(
        _NUM_WORKERS, n_chunks, _ROWS_PER_CHUNK * S
    ).astype(jnp.int32)
    return _make_gather(NB, S, V, D)(idx, coordinates)


# speedup vs baseline: 9.3849x; 1.7808x over previous
import functools
import math

import jax
import jax.numpy as jnp
from jax import lax
from jax.experimental import pallas as pl
from jax.experimental.pallas import tpu as pltpu
from jax.experimental.pallas import tpu_sc as plsc

_NUM_CORES = 2
_NUM_SUBCORES = 16
_NUM_WORKERS = _NUM_CORES * _NUM_SUBCORES
_LANES = 16

_NBUF = 5
_LEAD = 3


def _make_gather(NB, S, V, D):
    b_per_w = NB // _NUM_WORKERS
    n_groups = S // _NBUF
    scale = math.sqrt(D)
    mesh = plsc.VectorSubcoreMesh(core_axis_name="c", subcore_axis_name="s")

    @functools.partial(
        pl.kernel,
        out_type=jax.ShapeDtypeStruct((S, NB, D), jnp.float32),
        mesh=mesh,
        scratch_types=[
            pltpu.VMEM((S, b_per_w), jnp.int32),
            [pltpu.VMEM((b_per_w, D), jnp.float32) for _ in range(_NBUF)],
            [pltpu.SemaphoreType.DMA for _ in range(_NBUF)],
            [pltpu.SemaphoreType.DMA for _ in range(_NBUF)],
        ],
    )
    def gather_kernel(idx_hbm, table_hbm, out_hbm, idx_v, bufs, gsems, ssems):
        wid = lax.axis_index("s") * _NUM_CORES + lax.axis_index("c")
        b_base = wid * b_per_w
        pltpu.sync_copy(idx_hbm.at[:, pl.ds(b_base, b_per_w)], idx_v)

        def fire_gather(t, b):
            pltpu.async_copy(table_hbm.at[idx_v.at[t]], bufs[b], gsems[b])

        def wait_scatter(b):
            pltpu.make_async_copy(
                bufs[b], out_hbm.at[0, pl.ds(0, b_per_w)], ssems[b]
            ).wait()

        for b in range(_LEAD):
            fire_gather(b, b)

        def group_body(g, carry):
            for b in range(_NBUF):
                t = g * _NBUF + b
                pltpu.make_async_copy(
                    table_hbm.at[idx_v.at[t]], bufs[b], gsems[b]
                ).wait()

                @plsc.parallel_loop(0, b_per_w, unroll=4)
                def scale_rows(r):
                    for k in range(D // _LANES):
                        sl = pl.ds(k * _LANES, _LANES)
                        bufs[b][r, sl] = bufs[b][r, sl] * scale

                pltpu.async_copy(
                    bufs[b], out_hbm.at[t, pl.ds(b_base, b_per_w)], ssems[b]
                )

                bb = (b + _LEAD) % _NBUF

                @pl.when(t >= _NBUF - _LEAD)
                def _():
                    wait_scatter(bb)

                @pl.when(t + _LEAD < S)
                def _():
                    fire_gather(t + _LEAD, bb)

            return carry

        lax.fori_loop(0, n_groups, group_body, 0)

        for t in range(S - (_NBUF - _LEAD), S):
            wait_scatter(t % _NBUF)

    return gather_kernel


@jax.jit
def kernel(input, coordinates):
    V, D = coordinates.shape
    NB, S = input.shape
    idx_t = jnp.transpose(input.astype(jnp.int32), (1, 0))
    out_t = _make_gather(NB, S, V, D)(idx_t, coordinates)
    return jnp.transpose(out_t, (1, 0, 2))
